# baseline (speedup 1.0000x reference)
import functools

import jax
import jax.numpy as jnp
from jax.experimental import pallas as pl
from jax.experimental.pallas import tpu as pltpu

ATOM = 256
HID = 256


def _mm_kernel(x_ref, w_ref, o_ref, *, relu):
    acc = jnp.dot(x_ref[...], w_ref[...], preferred_element_type=jnp.float32)
    o_ref[...] = jnp.maximum(acc, 0.0) if relu else acc


def _matmul(x, w, relu=False, block_rows=1000):
    m, k = x.shape
    k2, n = w.shape
    assert k == k2 and m % block_rows == 0
    return pl.pallas_call(
        functools.partial(_mm_kernel, relu=relu),
        grid=(m // block_rows,),
        in_specs=[
            pl.BlockSpec((block_rows, k), lambda i: (i, 0)),
            pl.BlockSpec((k, n), lambda i: (0, 0)),
        ],
        out_specs=pl.BlockSpec((block_rows, n), lambda i: (i, 0)),
        out_shape=jax.ShapeDtypeStruct((m, n), jnp.float32),
    )(x, w)


def _mm2_kernel(x_ref, y_ref, wx_ref, wy_ref, o_ref, acc_ref, *, nsteps):
    @pl.when(pl.program_id(0) == 0)
    def _():
        acc_ref[...] = jnp.zeros_like(acc_ref)

    part = jnp.dot(x_ref[...], wx_ref[...], preferred_element_type=jnp.float32)
    part += jnp.dot(y_ref[...], wy_ref[...], preferred_element_type=jnp.float32)
    acc_ref[...] += jnp.sum(jnp.maximum(part, 0.0), axis=0, keepdims=True)

    @pl.when(pl.program_id(0) == nsteps - 1)
    def _():
        o_ref[...] = acc_ref[...]


def _final_stage(x, node_m, wx, wy, block_rows=1000):
    m = x.shape[0]
    nsteps = m // block_rows
    out = pl.pallas_call(
        functools.partial(_mm2_kernel, nsteps=nsteps),
        grid=(nsteps,),
        in_specs=[
            pl.BlockSpec((block_rows, ATOM), lambda i: (i, 0)),
            pl.BlockSpec((block_rows, HID), lambda i: (i, 0)),
            pl.BlockSpec((ATOM, HID), lambda i: (0, 0)),
            pl.BlockSpec((HID, HID), lambda i: (0, 0)),
        ],
        out_specs=pl.BlockSpec((1, HID), lambda i: (0, 0)),
        out_shape=jax.ShapeDtypeStruct((1, HID), jnp.float32),
        scratch_shapes=[pltpu.VMEM((1, HID), jnp.float32)],
    )(x, node_m, wx, wy)
    return out[0]


def kernel(node_feature, edge_featrue, edge_index, W_i, W_m, W_a, mpnn_hop):
    N = node_feature.shape[0]
    s = edge_index[0].astype(jnp.int32)
    e = edge_index[1].astype(jnp.int32)

    W_i1 = W_i[:, :ATOM].T
    W_i2 = W_i[:, ATOM:].T
    W_mT = W_m.T
    W_a1 = W_a[:, :ATOM].T
    W_a2 = W_a[:, ATOM:].T

    P = _matmul(node_feature, W_i1)
    Q = _matmul(edge_featrue, W_i2, block_rows=2000)

    rk = e * N + s
    order = jnp.argsort(rk)
    ss = s[order]
    ee = e[order]
    rks = rk[order]
    kq = ss * N + ee
    lo = jnp.searchsorted(rks, kq, side='left')
    hi = jnp.searchsorted(rks, kq, side='right')
    cnt = hi - lo

    h_sorted = jax.nn.relu(P[ss] + Q[order])
    agg = jax.ops.segment_sum(h_sorted, ee, num_segments=N)
    A_ = _matmul(agg, W_mT)
    R = jax.nn.relu(A_)

    node_m = jax.ops.segment_sum(R[ss], ee, num_segments=N)

    csum = jnp.concatenate([jnp.zeros((1, HID), jnp.float32), jnp.cumsum(h_sorted, axis=0)], axis=0)
    sub = csum[hi] - csum[lo]
    T = _matmul(sub, W_mT, block_rows=2000)
    corr = jnp.where((cnt > 0)[:, None], jax.nn.relu(A_[ss] - T) - R[ss], 0.0)
    node_m = node_m + jax.ops.segment_sum(corr, ee, num_segments=N)

    return _final_stage(node_feature, node_m, W_a1, W_a2)


# speedup vs baseline: 1.1694x; 1.1694x over previous
import functools

import jax
import jax.numpy as jnp
from jax import lax
from jax.experimental import pallas as pl
from jax.experimental.pallas import tpu as pltpu
from jax.experimental.pallas import tpu_sc as plsc

ATOM = 256
HID = 256
NSUB = 16
L = 16
NW = 32
BATCH = 64
TROWS = 320
NPAD = NW * TROWS
ACC_B = 328
DUMP_B = 320
SROWS = 160
ACC_D = 168
DUMP_D = 160
MLEN_A = 120
MLEN_B = 80
IDXLEN = 80
SCALEN = 32


def _mm_kernel(x_ref, w_ref, o_ref, *, relu):
    acc = jnp.dot(x_ref[...], w_ref[...], preferred_element_type=jnp.float32)
    o_ref[...] = jnp.maximum(acc, 0.0) if relu else acc


def _matmul(x, w, relu=False, block_rows=1000):
    m, k = x.shape
    k2, n = w.shape
    assert k == k2 and m % block_rows == 0
    return pl.pallas_call(
        functools.partial(_mm_kernel, relu=relu),
        grid=(m // block_rows,),
        in_specs=[
            pl.BlockSpec((block_rows, k), lambda i: (i, 0)),
            pl.BlockSpec((k, n), lambda i: (0, 0)),
        ],
        out_specs=pl.BlockSpec((block_rows, n), lambda i: (i, 0)),
        out_shape=jax.ShapeDtypeStruct((m, n), jnp.float32),
    )(x, w)


def _mm_ar_kernel(x_ref, w_ref, a_ref, r_ref):
    acc = jnp.dot(x_ref[...], w_ref[...], preferred_element_type=jnp.float32)
    a_ref[...] = acc
    r_ref[...] = jnp.maximum(acc, 0.0)


def _matmul_a_r(x, w, block_rows=1000):
    m, k = x.shape
    _, n = w.shape
    return pl.pallas_call(
        _mm_ar_kernel,
        grid=(m // block_rows,),
        in_specs=[
            pl.BlockSpec((block_rows, k), lambda i: (i, 0)),
            pl.BlockSpec((k, n), lambda i: (0, 0)),
        ],
        out_specs=[
            pl.BlockSpec((block_rows, n), lambda i: (i, 0)),
            pl.BlockSpec((block_rows, n), lambda i: (i, 0)),
        ],
        out_shape=[
            jax.ShapeDtypeStruct((m, n), jnp.float32),
            jax.ShapeDtypeStruct((m, n), jnp.float32),
        ],
    )(x, w)


def _mm2_kernel(x_ref, y_ref, wx_ref, wy_ref, o_ref, acc_ref, *, nsteps):
    @pl.when(pl.program_id(0) == 0)
    def _():
        acc_ref[...] = jnp.zeros_like(acc_ref)

    part = jnp.dot(x_ref[...], wx_ref[...], preferred_element_type=jnp.float32)
    part += jnp.dot(y_ref[...], wy_ref[...], preferred_element_type=jnp.float32)
    acc_ref[...] += jnp.sum(jnp.maximum(part, 0.0), axis=0, keepdims=True)

    @pl.when(pl.program_id(0) == nsteps - 1)
    def _():
        o_ref[...] = acc_ref[...]


def _final_stage(x, node_m, wx, wy, block_rows=1000):
    m = x.shape[0]
    nsteps = m // block_rows
    out = pl.pallas_call(
        functools.partial(_mm2_kernel, nsteps=nsteps),
        grid=(nsteps,),
        in_specs=[
            pl.BlockSpec((block_rows, ATOM), lambda i: (i, 0)),
            pl.BlockSpec((block_rows, HID), lambda i: (i, 0)),
            pl.BlockSpec((ATOM, HID), lambda i: (0, 0)),
            pl.BlockSpec((HID, HID), lambda i: (0, 0)),
        ],
        out_specs=pl.BlockSpec((1, HID), lambda i: (0, 0)),
        out_shape=jax.ShapeDtypeStruct((1, HID), jnp.float32),
        scratch_shapes=[pltpu.VMEM((1, HID), jnp.float32)],
    )(x, node_m, wx, wy)
    return out[0]


def _iota16():
    return lax.iota(jnp.int32, L)


def _lane(ref, lane):
    return ref[pl.ds(lane, L)][0]


def _zero_rows(acc, nrows):
    zv = jnp.zeros((L,), jnp.float32)

    def zbody(r, _):
        for ck in range(HID // L):
            acc[r, pl.ds(ck * L, L)] = zv
        return 0

    lax.fori_loop(0, nrows, zbody, 0)


def _make_idx_loc(idx_ee, idx_loc, b, tstart, tend, nbase, dump):
    def chunk_body(k, _):
        pos = b + k * L + _iota16()
        eev = idx_ee[pl.ds(k * L, L)]
        mask = (pos >= tstart) & (pos < tend)
        idx_loc[pl.ds(k * L, L)] = jnp.where(mask, eev - nbase, dump)
        return 0

    lax.fori_loop(0, BATCH // L, chunk_body, 0)


def _stage_b_body(P, Q, ss, orr, ee, meta, out,
                  mvec, idx_ss, idx_or, idx_ee, idx_loc, acc,
                  sem1, sem2, semi):
    c = lax.axis_index("c")
    sid = lax.axis_index("s")
    w = c * NSUB + sid
    pltpu.sync_copy(meta, mvec)
    tstart = _lane(mvec, w)
    tend = _lane(mvec, w + 1)
    nbase = w * TROWS
    astart = (tstart // 8) * 8
    nb = (tend - astart + BATCH - 1) // BATCH

    _zero_rows(acc, ACC_B)

    def run(bufP, bufQ):
        def batch_body(t, _):
            b = astart + t * BATCH
            c1 = pltpu.async_copy(ss.at[pl.ds(b, BATCH)], idx_ss, semi)
            c2 = pltpu.async_copy(orr.at[pl.ds(b, BATCH)], idx_or, semi)
            c3 = pltpu.async_copy(ee.at[pl.ds(b, BATCH)], idx_ee, semi)
            c1.wait()
            c2.wait()
            c3.wait()
            g1 = pltpu.async_copy(P.at[idx_ss], bufP, sem1)
            g2 = pltpu.async_copy(Q.at[idx_or], bufQ, sem2)
            _make_idx_loc(idx_ee, idx_loc, b, tstart, tend, nbase, DUMP_B)
            g1.wait()
            g2.wait()

            def row_body(r, _):
                dl = _lane(idx_loc, r)
                for ck in range(HID // L):
                    sl = pl.ds(ck * L, L)
                    h = jnp.maximum(bufP[r, sl] + bufQ[r, sl], 0.0)
                    plsc.addupdate(acc.at[dl, sl], h)
                return 0

            lax.fori_loop(0, BATCH, row_body, 0)
            return 0

        lax.fori_loop(0, nb, batch_body, 0)

    pl.run_scoped(run,
                  pltpu.VMEM((BATCH, HID), jnp.float32),
                  pltpu.VMEM((BATCH, HID), jnp.float32))
    pltpu.sync_copy(acc.at[pl.ds(0, TROWS)], out.at[pl.ds(nbase, TROWS)])


def _stage_b(P, Q, ss_pad, order_pad, ee_pad, meta):
    mesh = plsc.VectorSubcoreMesh(core_axis_name="c", subcore_axis_name="s")
    return pl.kernel(
        _stage_b_body,
        out_type=jax.ShapeDtypeStruct((NPAD, HID), jnp.float32),
        mesh=mesh,
        scratch_types=[
            pltpu.VMEM((MLEN_A,), jnp.int32),
            pltpu.VMEM((BATCH,), jnp.int32),
            pltpu.VMEM((BATCH,), jnp.int32),
            pltpu.VMEM((BATCH,), jnp.int32),
            pltpu.VMEM((IDXLEN,), jnp.int32),
            pltpu.VMEM((ACC_B, HID), jnp.float32),
            pltpu.SemaphoreType.DMA,
            pltpu.SemaphoreType.DMA,
            pltpu.SemaphoreType.DMA,
        ],
    )(P, Q, ss_pad, order_pad, ee_pad, meta)


def _stage_d_body(Rm, A_, P, Q, Wm, ss, orr, ee, pp, plo, phi, meta, metb, out,
                  mvec, mvecb, idx_ss, idx_ee, idx_loc, sca, acc,
                  sem1, semi):
    c = lax.axis_index("c")
    sid = lax.axis_index("s")
    w = c * NSUB + sid
    pltpu.sync_copy(meta, mvec)
    pltpu.sync_copy(metb, mvecb)

    for p in (0, 1):
        seg = 2 * w + p
        tstart = _lane(mvec, 33 + seg)
        tend = _lane(mvec, 33 + seg + 1)
        pstart = _lane(mvecb, seg)
        pend = _lane(mvecb, seg + 1)
        nbase = seg * SROWS
        astart = (tstart // 8) * 8
        nb = (tend - astart + BATCH - 1) // BATCH

        _zero_rows(acc, ACC_D)

        def fast(bufR):
            def batch_body(t, _):
                b = astart + t * BATCH
                c1 = pltpu.async_copy(ss.at[pl.ds(b, BATCH)], idx_ss, semi)
                c2 = pltpu.async_copy(ee.at[pl.ds(b, BATCH)], idx_ee, semi)
                c1.wait()
                c2.wait()
                g1 = pltpu.async_copy(Rm.at[idx_ss], bufR, sem1)
                _make_idx_loc(idx_ee, idx_loc, b, tstart, tend, nbase, DUMP_D)
                g1.wait()

                def row_body(r, _):
                    dl = _lane(idx_loc, r)
                    for ck in range(HID // L):
                        sl = pl.ds(ck * L, L)
                        plsc.addupdate(acc.at[dl, sl], bufR[r, sl])
                    return 0

                lax.fori_loop(0, BATCH, row_body, 0)
                return 0

            lax.fori_loop(0, nb, batch_body, 0)

        pl.run_scoped(fast, pltpu.VMEM((BATCH, HID), jnp.float32))

        @pl.when(pend > pstart)
        def _():
            def slow(wmv, rowP, rowQ, rowS):
                pltpu.sync_copy(Wm, wmv)

                def paired_body(j, carry):
                    jb = (j // L) * L
                    jl = j - jb
                    pltpu.sync_copy(pp.at[pl.ds(jb, L)], sca.at[pl.ds(0, L)])
                    t_pos = _lane(sca, jl)
                    pltpu.sync_copy(plo.at[pl.ds(jb, L)], sca.at[pl.ds(0, L)])
                    r_lo = _lane(sca, jl)
                    pltpu.sync_copy(phi.at[pl.ds(jb, L)], sca.at[pl.ds(0, L)])
                    r_hi = _lane(sca, jl)
                    tb = (t_pos // L) * L
                    tl = t_pos - tb
                    pltpu.sync_copy(ss.at[pl.ds(tb, L)], sca.at[pl.ds(0, L)])
                    s_t = _lane(sca, tl)
                    pltpu.sync_copy(ee.at[pl.ds(tb, L)], sca.at[pl.ds(0, L)])
                    e_t = _lane(sca, tl)

                    for ck in range(HID // L):
                        rowS[pl.ds(ck * L, L)] = jnp.zeros((L,), jnp.float32)

                    def rev_body(r, carry2):
                        rb = (r // L) * L
                        rl = r - rb
                        pltpu.sync_copy(ss.at[pl.ds(rb, L)], sca.at[pl.ds(0, L)])
                        s_r = _lane(sca, rl)
                        pltpu.sync_copy(orr.at[pl.ds(rb, L)], sca.at[pl.ds(0, L)])
                        o_r = _lane(sca, rl)
                        pltpu.sync_copy(P.at[pl.ds(s_r, 1)], rowP)
                        pltpu.sync_copy(Q.at[pl.ds(o_r, 1)], rowQ)
                        for ck in range(HID // L):
                            sl = pl.ds(ck * L, L)
                            h = jnp.maximum(rowP[0, sl] + rowQ[0, sl], 0.0)
                            rowS[sl] = rowS[sl] + h
                        return carry2

                    lax.fori_loop(r_lo, r_hi, rev_body, 0)

                    def mv_body(k, accs):
                        sk = jnp.zeros((L,), jnp.float32) + _lane(rowS, k)
                        return tuple(
                            accs[ck] + sk * wmv[k, pl.ds(ck * L, L)]
                            for ck in range(HID // L)
                        )

                    accs0 = tuple(jnp.zeros((L,), jnp.float32)
                                  for _ in range(HID // L))
                    accs = lax.fori_loop(0, HID, mv_body, accs0)

                    pltpu.sync_copy(A_.at[pl.ds(s_t, 1)], rowP)
                    pltpu.sync_copy(Rm.at[pl.ds(s_t, 1)], rowQ)
                    dl = e_t - nbase
                    for ck in range(HID // L):
                        sl = pl.ds(ck * L, L)
                        delta = (jnp.maximum(rowP[0, sl] - accs[ck], 0.0)
                                 - rowQ[0, sl])
                        plsc.addupdate(acc.at[dl, sl], delta)
                    return carry

                lax.fori_loop(pstart, pend, paired_body, 0)

            pl.run_scoped(slow,
                          pltpu.VMEM((HID, HID), jnp.float32),
                          pltpu.VMEM((1, HID), jnp.float32),
                          pltpu.VMEM((1, HID), jnp.float32),
                          pltpu.VMEM((HID + L,), jnp.float32))

        pltpu.sync_copy(acc.at[pl.ds(0, SROWS)], out.at[pl.ds(nbase, SROWS)])


def _stage_d(Rm, A_, P, Q, Wm_T, ss_pad, order_pad, ee_pad, pp, plo, phi, meta,
             metb):
    mesh = plsc.VectorSubcoreMesh(core_axis_name="c", subcore_axis_name="s")
    return pl.kernel(
        _stage_d_body,
        out_type=jax.ShapeDtypeStruct((NPAD, HID), jnp.float32),
        mesh=mesh,
        scratch_types=[
            pltpu.VMEM((MLEN_A,), jnp.int32),
            pltpu.VMEM((MLEN_B,), jnp.int32),
            pltpu.VMEM((BATCH,), jnp.int32),
            pltpu.VMEM((BATCH,), jnp.int32),
            pltpu.VMEM((IDXLEN,), jnp.int32),
            pltpu.VMEM((SCALEN,), jnp.int32),
            pltpu.VMEM((ACC_D, HID), jnp.float32),
            pltpu.SemaphoreType.DMA,
            pltpu.SemaphoreType.DMA,
        ],
    )(Rm, A_, P, Q, Wm_T, ss_pad, order_pad, ee_pad, pp, plo, phi, meta, metb)


def kernel(node_feature, edge_featrue, edge_index, W_i, W_m, W_a, mpnn_hop):
    N = node_feature.shape[0]
    E = edge_index.shape[1]
    s = edge_index[0].astype(jnp.int32)
    e = edge_index[1].astype(jnp.int32)

    W_i1 = W_i[:, :ATOM].T
    W_i2 = W_i[:, ATOM:].T
    W_mT = W_m.T
    W_a1 = W_a[:, :ATOM].T
    W_a2 = W_a[:, ATOM:].T

    P = _matmul(node_feature, W_i1)
    Q = _matmul(edge_featrue, W_i2, block_rows=2000)

    rk = e * N + s
    order = jnp.argsort(rk)
    ss = s[order]
    ee = e[order]
    rks = rk[order]
    kq = ss * N + ee
    lo = jnp.searchsorted(rks, kq, side='left').astype(jnp.int32)
    hi = jnp.searchsorted(rks, kq, side='right').astype(jnp.int32)
    cnt = hi - lo

    eb32 = jnp.searchsorted(
        ee, jnp.arange(33, dtype=jnp.int32) * TROWS).astype(jnp.int32)
    eb64 = jnp.searchsorted(
        ee, jnp.arange(65, dtype=jnp.int32) * SROWS).astype(jnp.int32)

    paired = cnt > 0
    (pp,) = jnp.nonzero(paired, size=E, fill_value=E)
    pp = pp.astype(jnp.int32)
    pe = jnp.where(paired, ee, NPAD)[pp.clip(0, E - 1)]
    pe = jnp.where(pp < E, pe, NPAD)
    peb64 = jnp.searchsorted(
        pe, jnp.arange(65, dtype=jnp.int32) * SROWS).astype(jnp.int32)
    plo = lo[pp.clip(0, E - 1)]
    phi = hi[pp.clip(0, E - 1)]

    pad_i = jnp.zeros((128,), jnp.int32)
    ss_pad = jnp.concatenate([ss, pad_i])
    ee_pad = jnp.concatenate([ee, pad_i])
    order_pad = jnp.concatenate([order.astype(jnp.int32), pad_i])
    pp_pad = jnp.concatenate([pp, pad_i])
    plo_pad = jnp.concatenate([plo, pad_i])
    phi_pad = jnp.concatenate([phi, pad_i])

    meta = jnp.concatenate(
        [eb32, eb64, jnp.zeros((MLEN_A - 98,), jnp.int32)])
    metb = jnp.concatenate([peb64, jnp.zeros((MLEN_B - 65,), jnp.int32)])

    agg = _stage_b(P, Q, ss_pad, order_pad, ee_pad, meta)[:N]
    A_, Rm = _matmul_a_r(agg, W_mT)
    node_m = _stage_d(Rm, A_, P, Q, W_mT, ss_pad, order_pad, ee_pad,
                      pp_pad, plo_pad, phi_pad, meta, metb)[:N]

    return _final_stage(node_feature, node_m, W_a1, W_a2)


# speedup vs baseline: 2.0931x; 1.7899x over previous
import functools

import jax
import jax.numpy as jnp
from jax import lax
from jax.experimental import pallas as pl
from jax.experimental.pallas import tpu as pltpu
from jax.experimental.pallas import tpu_sc as plsc

ATOM = 256
HID = 256
NSUB = 16
L = 16
NW = 32
BATCH = 64
TROWS = 320
NPAD = NW * TROWS
ACC_B = 328
DUMP_B = 320
SROWS = 160
ACC_D = 168
DUMP_D = 160
MLEN_A = 120
MLEN_B = 80
IDXLEN = 80
SCALEN = 32


def _mm_kernel(x_ref, w_ref, o_ref, *, relu):
    acc = jnp.dot(x_ref[...], w_ref[...], preferred_element_type=jnp.float32)
    o_ref[...] = jnp.maximum(acc, 0.0) if relu else acc


def _matmul(x, w, relu=False, block_rows=1000):
    m, k = x.shape
    k2, n = w.shape
    assert k == k2 and m % block_rows == 0
    return pl.pallas_call(
        functools.partial(_mm_kernel, relu=relu),
        grid=(m // block_rows,),
        in_specs=[
            pl.BlockSpec((block_rows, k), lambda i: (i, 0)),
            pl.BlockSpec((k, n), lambda i: (0, 0)),
        ],
        out_specs=pl.BlockSpec((block_rows, n), lambda i: (i, 0)),
        out_shape=jax.ShapeDtypeStruct((m, n), jnp.float32),
    )(x, w)


def _mm_ar_kernel(x_ref, w_ref, a_ref, r_ref):
    acc = jnp.dot(x_ref[...], w_ref[...], preferred_element_type=jnp.float32)
    a_ref[...] = acc
    r_ref[...] = jnp.maximum(acc, 0.0)


def _matmul_a_r(x, w, block_rows=1000):
    m, k = x.shape
    _, n = w.shape
    return pl.pallas_call(
        _mm_ar_kernel,
        grid=(m // block_rows,),
        in_specs=[
            pl.BlockSpec((block_rows, k), lambda i: (i, 0)),
            pl.BlockSpec((k, n), lambda i: (0, 0)),
        ],
        out_specs=[
            pl.BlockSpec((block_rows, n), lambda i: (i, 0)),
            pl.BlockSpec((block_rows, n), lambda i: (i, 0)),
        ],
        out_shape=[
            jax.ShapeDtypeStruct((m, n), jnp.float32),
            jax.ShapeDtypeStruct((m, n), jnp.float32),
        ],
    )(x, w)


def _mm2_kernel(x_ref, y_ref, wx_ref, wy_ref, o_ref, acc_ref, *, nsteps):
    @pl.when(pl.program_id(0) == 0)
    def _():
        acc_ref[...] = jnp.zeros_like(acc_ref)

    part = jnp.dot(x_ref[...], wx_ref[...], preferred_element_type=jnp.float32)
    part += jnp.dot(y_ref[...], wy_ref[...], preferred_element_type=jnp.float32)
    acc_ref[...] += jnp.sum(jnp.maximum(part, 0.0), axis=0, keepdims=True)

    @pl.when(pl.program_id(0) == nsteps - 1)
    def _():
        o_ref[...] = acc_ref[...]


def _final_stage(x, node_m, wx, wy, block_rows=1000):
    m = x.shape[0]
    nsteps = m // block_rows
    out = pl.pallas_call(
        functools.partial(_mm2_kernel, nsteps=nsteps),
        grid=(nsteps,),
        in_specs=[
            pl.BlockSpec((block_rows, ATOM), lambda i: (i, 0)),
            pl.BlockSpec((block_rows, HID), lambda i: (i, 0)),
            pl.BlockSpec((ATOM, HID), lambda i: (0, 0)),
            pl.BlockSpec((HID, HID), lambda i: (0, 0)),
        ],
        out_specs=pl.BlockSpec((1, HID), lambda i: (0, 0)),
        out_shape=jax.ShapeDtypeStruct((1, HID), jnp.float32),
        scratch_shapes=[pltpu.VMEM((1, HID), jnp.float32)],
    )(x, node_m, wx, wy)
    return out[0]


def _iota16():
    return lax.iota(jnp.int32, L)


def _lane(ref, lane):
    return ref[pl.ds(lane, L)][0]


def _zero_rows(acc, nrows):
    zv = jnp.zeros((L,), jnp.float32)

    def zbody(r, _):
        for ck in range(HID // L):
            acc[r, pl.ds(ck * L, L)] = zv
        return 0

    lax.fori_loop(0, nrows, zbody, 0)


def _make_idx_loc(idx_ee, idx_loc, b, tstart, tend, nbase, dump):
    def chunk_body(k, _):
        pos = b + k * L + _iota16()
        eev = idx_ee[pl.ds(k * L, L)]
        mask = (pos >= tstart) & (pos < tend)
        idx_loc[pl.ds(k * L, L)] = jnp.where(mask, eev - nbase, dump)
        return 0

    lax.fori_loop(0, BATCH // L, chunk_body, 0)


def _stage_b_body(P, Q, ss, orr, ee, meta, out,
                  mvec, idx_ss, idx_or, idx_ee, idx_loc, acc,
                  sem1, sem2, semi):
    c = lax.axis_index("c")
    sid = lax.axis_index("s")
    w = c * NSUB + sid
    pltpu.sync_copy(meta, mvec)
    tstart = _lane(mvec, w)
    tend = _lane(mvec, w + 1)
    nbase = w * TROWS
    astart = (tstart // 8) * 8
    nb = (tend - astart + BATCH - 1) // BATCH

    _zero_rows(acc, ACC_B)

    def run(bufP, bufQ):
        def batch_body(t, _):
            b = astart + t * BATCH
            c1 = pltpu.async_copy(ss.at[pl.ds(b, BATCH)], idx_ss, semi)
            c2 = pltpu.async_copy(orr.at[pl.ds(b, BATCH)], idx_or, semi)
            c3 = pltpu.async_copy(ee.at[pl.ds(b, BATCH)], idx_ee, semi)
            c1.wait()
            c2.wait()
            c3.wait()
            g1 = pltpu.async_copy(P.at[idx_ss], bufP, sem1)
            g2 = pltpu.async_copy(Q.at[idx_or], bufQ, sem2)
            _make_idx_loc(idx_ee, idx_loc, b, tstart, tend, nbase, DUMP_B)
            g1.wait()
            g2.wait()

            def row_body(r, _):
                dl = _lane(idx_loc, r)
                for ck in range(HID // L):
                    sl = pl.ds(ck * L, L)
                    h = jnp.maximum(bufP[r, sl] + bufQ[r, sl], 0.0)
                    plsc.addupdate(acc.at[dl, sl], h)
                return 0

            lax.fori_loop(0, BATCH, row_body, 0)
            return 0

        lax.fori_loop(0, nb, batch_body, 0)

    pl.run_scoped(run,
                  pltpu.VMEM((BATCH, HID), jnp.float32),
                  pltpu.VMEM((BATCH, HID), jnp.float32))
    pltpu.sync_copy(acc.at[pl.ds(0, TROWS)], out.at[pl.ds(nbase, TROWS)])


def _stage_b(P, Q, ss_pad, order_pad, ee_pad, meta):
    mesh = plsc.VectorSubcoreMesh(core_axis_name="c", subcore_axis_name="s")
    return pl.kernel(
        _stage_b_body,
        out_type=jax.ShapeDtypeStruct((NPAD, HID), jnp.float32),
        mesh=mesh,
        scratch_types=[
            pltpu.VMEM((MLEN_A,), jnp.int32),
            pltpu.VMEM((BATCH,), jnp.int32),
            pltpu.VMEM((BATCH,), jnp.int32),
            pltpu.VMEM((BATCH,), jnp.int32),
            pltpu.VMEM((IDXLEN,), jnp.int32),
            pltpu.VMEM((ACC_B, HID), jnp.float32),
            pltpu.SemaphoreType.DMA,
            pltpu.SemaphoreType.DMA,
            pltpu.SemaphoreType.DMA,
        ],
    )(P, Q, ss_pad, order_pad, ee_pad, meta)


def _stage_d_body(Rm, A_, P, Q, Wm, ss, orr, ee, pp, plo, phi, meta, metb, out,
                  mvec, mvecb, idx_ss, idx_ee, idx_loc, sca, acc,
                  sem1, semi):
    c = lax.axis_index("c")
    sid = lax.axis_index("s")
    w = c * NSUB + sid
    pltpu.sync_copy(meta, mvec)
    pltpu.sync_copy(metb, mvecb)

    for p in (0, 1):
        seg = 2 * w + p
        tstart = _lane(mvec, 33 + seg)
        tend = _lane(mvec, 33 + seg + 1)
        pstart = _lane(mvecb, seg)
        pend = _lane(mvecb, seg + 1)
        nbase = seg * SROWS
        astart = (tstart // 8) * 8
        nb = (tend - astart + BATCH - 1) // BATCH

        _zero_rows(acc, ACC_D)

        def fast(bufR):
            def batch_body(t, _):
                b = astart + t * BATCH
                c1 = pltpu.async_copy(ss.at[pl.ds(b, BATCH)], idx_ss, semi)
                c2 = pltpu.async_copy(ee.at[pl.ds(b, BATCH)], idx_ee, semi)
                c1.wait()
                c2.wait()
                g1 = pltpu.async_copy(Rm.at[idx_ss], bufR, sem1)
                _make_idx_loc(idx_ee, idx_loc, b, tstart, tend, nbase, DUMP_D)
                g1.wait()

                def row_body(r, _):
                    dl = _lane(idx_loc, r)
                    for ck in range(HID // L):
                        sl = pl.ds(ck * L, L)
                        plsc.addupdate(acc.at[dl, sl], bufR[r, sl])
                    return 0

                lax.fori_loop(0, BATCH, row_body, 0)
                return 0

            lax.fori_loop(0, nb, batch_body, 0)

        pl.run_scoped(fast, pltpu.VMEM((BATCH, HID), jnp.float32))

        @pl.when(pend > pstart)
        def _():
            def slow(wmv, rowP, rowQ, rowS):
                pltpu.sync_copy(Wm, wmv)

                def paired_body(j, carry):
                    jb = (j // L) * L
                    jl = j - jb
                    pltpu.sync_copy(pp.at[pl.ds(jb, L)], sca.at[pl.ds(0, L)])
                    t_pos = _lane(sca, jl)
                    pltpu.sync_copy(plo.at[pl.ds(jb, L)], sca.at[pl.ds(0, L)])
                    r_lo = _lane(sca, jl)
                    pltpu.sync_copy(phi.at[pl.ds(jb, L)], sca.at[pl.ds(0, L)])
                    r_hi = _lane(sca, jl)
                    tb = (t_pos // L) * L
                    tl = t_pos - tb
                    pltpu.sync_copy(ss.at[pl.ds(tb, L)], sca.at[pl.ds(0, L)])
                    s_t = _lane(sca, tl)
                    pltpu.sync_copy(ee.at[pl.ds(tb, L)], sca.at[pl.ds(0, L)])
                    e_t = _lane(sca, tl)

                    for ck in range(HID // L):
                        rowS[pl.ds(ck * L, L)] = jnp.zeros((L,), jnp.float32)

                    def rev_body(r, carry2):
                        rb = (r // L) * L
                        rl = r - rb
                        pltpu.sync_copy(ss.at[pl.ds(rb, L)], sca.at[pl.ds(0, L)])
                        s_r = _lane(sca, rl)
                        pltpu.sync_copy(orr.at[pl.ds(rb, L)], sca.at[pl.ds(0, L)])
                        o_r = _lane(sca, rl)
                        pltpu.sync_copy(P.at[pl.ds(s_r, 1)], rowP)
                        pltpu.sync_copy(Q.at[pl.ds(o_r, 1)], rowQ)
                        for ck in range(HID // L):
                            sl = pl.ds(ck * L, L)
                            h = jnp.maximum(rowP[0, sl] + rowQ[0, sl], 0.0)
                            rowS[sl] = rowS[sl] + h
                        return carry2

                    lax.fori_loop(r_lo, r_hi, rev_body, 0)

                    def mv_body(k, accs):
                        sk = jnp.zeros((L,), jnp.float32) + _lane(rowS, k)
                        return tuple(
                            accs[ck] + sk * wmv[k, pl.ds(ck * L, L)]
                            for ck in range(HID // L)
                        )

                    accs0 = tuple(jnp.zeros((L,), jnp.float32)
                                  for _ in range(HID // L))
                    accs = lax.fori_loop(0, HID, mv_body, accs0)

                    pltpu.sync_copy(A_.at[pl.ds(s_t, 1)], rowP)
                    pltpu.sync_copy(Rm.at[pl.ds(s_t, 1)], rowQ)
                    dl = e_t - nbase
                    for ck in range(HID // L):
                        sl = pl.ds(ck * L, L)
                        delta = (jnp.maximum(rowP[0, sl] - accs[ck], 0.0)
                                 - rowQ[0, sl])
                        plsc.addupdate(acc.at[dl, sl], delta)
                    return carry

                lax.fori_loop(pstart, pend, paired_body, 0)

            pl.run_scoped(slow,
                          pltpu.VMEM((HID, HID), jnp.float32),
                          pltpu.VMEM((1, HID), jnp.float32),
                          pltpu.VMEM((1, HID), jnp.float32),
                          pltpu.VMEM((HID + L,), jnp.float32))

        pltpu.sync_copy(acc.at[pl.ds(0, SROWS)], out.at[pl.ds(nbase, SROWS)])


def _stage_d(Rm, A_, P, Q, Wm_T, ss_pad, order_pad, ee_pad, pp, plo, phi, meta,
             metb):
    mesh = plsc.VectorSubcoreMesh(core_axis_name="c", subcore_axis_name="s")
    return pl.kernel(
        _stage_d_body,
        out_type=jax.ShapeDtypeStruct((NPAD, HID), jnp.float32),
        mesh=mesh,
        scratch_types=[
            pltpu.VMEM((MLEN_A,), jnp.int32),
            pltpu.VMEM((MLEN_B,), jnp.int32),
            pltpu.VMEM((BATCH,), jnp.int32),
            pltpu.VMEM((BATCH,), jnp.int32),
            pltpu.VMEM((IDXLEN,), jnp.int32),
            pltpu.VMEM((SCALEN,), jnp.int32),
            pltpu.VMEM((ACC_D, HID), jnp.float32),
            pltpu.SemaphoreType.DMA,
            pltpu.SemaphoreType.DMA,
        ],
    )(Rm, A_, P, Q, Wm_T, ss_pad, order_pad, ee_pad, pp, plo, phi, meta, metb)


def kernel(node_feature, edge_featrue, edge_index, W_i, W_m, W_a, mpnn_hop):
    N = node_feature.shape[0]
    E = edge_index.shape[1]
    s = edge_index[0].astype(jnp.int32)
    e = edge_index[1].astype(jnp.int32)

    W_i1 = W_i[:, :ATOM].T
    W_i2 = W_i[:, ATOM:].T
    W_mT = W_m.T
    W_a1 = W_a[:, :ATOM].T
    W_a2 = W_a[:, ATOM:].T

    P = _matmul(node_feature, W_i1)
    Q = _matmul(edge_featrue, W_i2, block_rows=2000)

    t_iota = jnp.arange(E, dtype=jnp.int32)
    pk = e * 16384 + s
    pks, order = lax.sort((pk, t_iota), num_keys=1)
    ee = pks >> 14
    ss = pks & 16383
    rks = ee * N + ss
    fk = ss * N + ee

    tag_keys = jnp.concatenate([rks * 4 + 1, fk * 4 + 0, fk * 4 + 2])
    tag_ids = jnp.concatenate(
        [jnp.full((E,), 2 * E, jnp.int32), t_iota, t_iota + E])
    cks, cids = lax.sort((tag_keys, tag_ids), num_keys=1)
    is_data = ((cks & 3) == 1).astype(jnp.int32)
    c0 = jnp.cumsum(is_data) - is_data
    lohi = jnp.zeros((2 * E + 1,), jnp.int32).at[cids].set(c0)
    lo = lohi[:E]
    hi = lohi[E:2 * E]
    paired = hi > lo

    pkey = jnp.where(paired, t_iota, t_iota + E)
    pkey_s, plo, phi, pe_s, pp = lax.sort((pkey, lo, hi, ee, t_iota),
                                          num_keys=1)
    pe = jnp.where(pkey_s < E, pe_s, NPAD)

    b32 = jnp.arange(33, dtype=jnp.int32) * TROWS
    b64 = jnp.arange(65, dtype=jnp.int32) * SROWS
    eb32 = jnp.sum(ee[None, :] < b32[:, None], axis=1).astype(jnp.int32)
    eb64 = jnp.sum(ee[None, :] < b64[:, None], axis=1).astype(jnp.int32)
    peb64 = jnp.sum(pe[None, :] < b64[:, None], axis=1).astype(jnp.int32)

    pad_i = jnp.zeros((128,), jnp.int32)
    ss_pad = jnp.concatenate([ss, pad_i])
    ee_pad = jnp.concatenate([ee, pad_i])
    order_pad = jnp.concatenate([order.astype(jnp.int32), pad_i])
    pp_pad = jnp.concatenate([pp, pad_i])
    plo_pad = jnp.concatenate([plo, pad_i])
    phi_pad = jnp.concatenate([phi, pad_i])

    meta = jnp.concatenate(
        [eb32, eb64, jnp.zeros((MLEN_A - 98,), jnp.int32)])
    metb = jnp.concatenate([peb64, jnp.zeros((MLEN_B - 65,), jnp.int32)])

    agg = _stage_b(P, Q, ss_pad, order_pad, ee_pad, meta)[:N]
    A_, Rm = _matmul_a_r(agg, W_mT)
    node_m = _stage_d(Rm, A_, P, Q, W_mT, ss_pad, order_pad, ee_pad,
                      pp_pad, plo_pad, phi_pad, meta, metb)[:N]

    return _final_stage(node_feature, node_m, W_a1, W_a2)


# speedup vs baseline: 2.9391x; 1.4042x over previous
import functools

import jax
import jax.numpy as jnp
from jax import lax
from jax.experimental import pallas as pl
from jax.experimental.pallas import tpu as pltpu
from jax.experimental.pallas import tpu_sc as plsc

ATOM = 256
HID = 256
NSUB = 16
L = 16
NW = 32
BATCH = 64
TROWS = 320
NPAD = NW * TROWS
ACC_B = 328
DUMP_B = 320
SROWS = 160
ACC_D = 168
DUMP_D = 160
MLEN_A = 120
MLEN_B = 80
IDXLEN = 80
SCALEN = 32


def _mm_kernel(x_ref, w_ref, o_ref, *, relu):
    acc = jnp.dot(x_ref[...], w_ref[...], preferred_element_type=jnp.float32)
    o_ref[...] = jnp.maximum(acc, 0.0) if relu else acc


def _matmul(x, w, relu=False, block_rows=1000):
    m, k = x.shape
    k2, n = w.shape
    assert k == k2 and m % block_rows == 0
    return pl.pallas_call(
        functools.partial(_mm_kernel, relu=relu),
        grid=(m // block_rows,),
        in_specs=[
            pl.BlockSpec((block_rows, k), lambda i: (i, 0)),
            pl.BlockSpec((k, n), lambda i: (0, 0)),
        ],
        out_specs=pl.BlockSpec((block_rows, n), lambda i: (i, 0)),
        out_shape=jax.ShapeDtypeStruct((m, n), jnp.float32),
    )(x, w)


def _mm_ar_kernel(x_ref, w_ref, a_ref, r_ref):
    acc = jnp.dot(x_ref[...], w_ref[...], preferred_element_type=jnp.float32)
    a_ref[...] = acc
    r_ref[...] = jnp.maximum(acc, 0.0)


def _matmul_a_r(x, w, block_rows=1000):
    m, k = x.shape
    _, n = w.shape
    return pl.pallas_call(
        _mm_ar_kernel,
        grid=(m // block_rows,),
        in_specs=[
            pl.BlockSpec((block_rows, k), lambda i: (i, 0)),
            pl.BlockSpec((k, n), lambda i: (0, 0)),
        ],
        out_specs=[
            pl.BlockSpec((block_rows, n), lambda i: (i, 0)),
            pl.BlockSpec((block_rows, n), lambda i: (i, 0)),
        ],
        out_shape=[
            jax.ShapeDtypeStruct((m, n), jnp.float32),
            jax.ShapeDtypeStruct((m, n), jnp.float32),
        ],
    )(x, w)


def _mm2_kernel(x_ref, y_ref, wx_ref, wy_ref, o_ref, acc_ref, *, nsteps):
    @pl.when(pl.program_id(0) == 0)
    def _():
        acc_ref[...] = jnp.zeros_like(acc_ref)

    part = jnp.dot(x_ref[...], wx_ref[...], preferred_element_type=jnp.float32)
    part += jnp.dot(y_ref[...], wy_ref[...], preferred_element_type=jnp.float32)
    acc_ref[...] += jnp.sum(jnp.maximum(part, 0.0), axis=0, keepdims=True)

    @pl.when(pl.program_id(0) == nsteps - 1)
    def _():
        o_ref[...] = acc_ref[...]


def _final_stage(x, node_m, wx, wy, block_rows=1000):
    m = x.shape[0]
    nsteps = m // block_rows
    out = pl.pallas_call(
        functools.partial(_mm2_kernel, nsteps=nsteps),
        grid=(nsteps,),
        in_specs=[
            pl.BlockSpec((block_rows, ATOM), lambda i: (i, 0)),
            pl.BlockSpec((block_rows, HID), lambda i: (i, 0)),
            pl.BlockSpec((ATOM, HID), lambda i: (0, 0)),
            pl.BlockSpec((HID, HID), lambda i: (0, 0)),
        ],
        out_specs=pl.BlockSpec((1, HID), lambda i: (0, 0)),
        out_shape=jax.ShapeDtypeStruct((1, HID), jnp.float32),
        scratch_shapes=[pltpu.VMEM((1, HID), jnp.float32)],
    )(x, node_m, wx, wy)
    return out[0]


def _iota16():
    return lax.iota(jnp.int32, L)


def _lane(ref, lane):
    return ref[pl.ds(lane, L)][0]


def _zero_rows(acc, nrows):
    zv = jnp.zeros((L,), jnp.float32)

    def zbody(r, _):
        for ck in range(HID // L):
            acc[r, pl.ds(ck * L, L)] = zv
        return 0

    lax.fori_loop(0, nrows, zbody, 0)


def _make_idx_loc(idx_ee, idx_loc, b, tstart, tend, nbase, dump):
    def chunk_body(k, _):
        pos = b + k * L + _iota16()
        eev = idx_ee[pl.ds(k * L, L)]
        mask = (pos >= tstart) & (pos < tend)
        idx_loc[pl.ds(k * L, L)] = jnp.where(mask, eev - nbase, dump)
        return 0

    lax.fori_loop(0, BATCH // L, chunk_body, 0)


def _stage_b_body(P, Q, ss, orr, ee, meta, out,
                  mvec, idx_ss, idx_or, idx_ee, idx_loc, acc,
                  sem1, sem2, semi):
    c = lax.axis_index("c")
    sid = lax.axis_index("s")
    w = c * NSUB + sid
    pltpu.sync_copy(meta, mvec)
    tstart = _lane(mvec, w)
    tend = _lane(mvec, w + 1)
    nbase = w * TROWS
    astart = (tstart // 8) * 8
    nb = (tend - astart + BATCH - 1) // BATCH

    _zero_rows(acc, ACC_B)

    def run(bufP, bufQ):
        def batch_body(t, _):
            b = astart + t * BATCH
            c1 = pltpu.async_copy(ss.at[pl.ds(b, BATCH)], idx_ss, semi)
            c2 = pltpu.async_copy(orr.at[pl.ds(b, BATCH)], idx_or, semi)
            c3 = pltpu.async_copy(ee.at[pl.ds(b, BATCH)], idx_ee, semi)
            c1.wait()
            c2.wait()
            c3.wait()
            g1 = pltpu.async_copy(P.at[idx_ss], bufP, sem1)
            g2 = pltpu.async_copy(Q.at[idx_or], bufQ, sem2)
            _make_idx_loc(idx_ee, idx_loc, b, tstart, tend, nbase, DUMP_B)
            g1.wait()
            g2.wait()

            def row_body(r, _):
                dl = _lane(idx_loc, r)
                for ck in range(HID // L):
                    sl = pl.ds(ck * L, L)
                    h = jnp.maximum(bufP[r, sl] + bufQ[r, sl], 0.0)
                    plsc.addupdate(acc.at[dl, sl], h)
                return 0

            lax.fori_loop(0, BATCH, row_body, 0)
            return 0

        lax.fori_loop(0, nb, batch_body, 0)

    pl.run_scoped(run,
                  pltpu.VMEM((BATCH, HID), jnp.float32),
                  pltpu.VMEM((BATCH, HID), jnp.float32))
    pltpu.sync_copy(acc.at[pl.ds(0, TROWS)], out.at[pl.ds(nbase, TROWS)])


def _stage_b(P, Q, ss_pad, order_pad, ee_pad, meta):
    mesh = plsc.VectorSubcoreMesh(core_axis_name="c", subcore_axis_name="s")
    return pl.kernel(
        _stage_b_body,
        out_type=jax.ShapeDtypeStruct((NPAD, HID), jnp.float32),
        mesh=mesh,
        scratch_types=[
            pltpu.VMEM((MLEN_A,), jnp.int32),
            pltpu.VMEM((BATCH,), jnp.int32),
            pltpu.VMEM((BATCH,), jnp.int32),
            pltpu.VMEM((BATCH,), jnp.int32),
            pltpu.VMEM((IDXLEN,), jnp.int32),
            pltpu.VMEM((ACC_B, HID), jnp.float32),
            pltpu.SemaphoreType.DMA,
            pltpu.SemaphoreType.DMA,
            pltpu.SemaphoreType.DMA,
        ],
    )(P, Q, ss_pad, order_pad, ee_pad, meta)


def _stage_d_body(Rm, A_, P, Q, Wm, ss, orr, ee, pp, plo, phi, meta, metb, out,
                  mvec, mvecb, idx_ss, idx_ee, idx_loc, sca, acc,
                  sem1, semi):
    c = lax.axis_index("c")
    sid = lax.axis_index("s")
    w = c * NSUB + sid
    pltpu.sync_copy(meta, mvec)
    pltpu.sync_copy(metb, mvecb)

    for p in (0, 1):
        seg = 2 * w + p
        tstart = _lane(mvec, 33 + seg)
        tend = _lane(mvec, 33 + seg + 1)
        pstart = _lane(mvecb, seg)
        pend = _lane(mvecb, seg + 1)
        nbase = seg * SROWS
        astart = (tstart // 8) * 8
        nb = (tend - astart + BATCH - 1) // BATCH

        _zero_rows(acc, ACC_D)

        def fast(bufR):
            def batch_body(t, _):
                b = astart + t * BATCH
                c1 = pltpu.async_copy(ss.at[pl.ds(b, BATCH)], idx_ss, semi)
                c2 = pltpu.async_copy(ee.at[pl.ds(b, BATCH)], idx_ee, semi)
                c1.wait()
                c2.wait()
                g1 = pltpu.async_copy(Rm.at[idx_ss], bufR, sem1)
                _make_idx_loc(idx_ee, idx_loc, b, tstart, tend, nbase, DUMP_D)
                g1.wait()

                def row_body(r, _):
                    dl = _lane(idx_loc, r)
                    for ck in range(HID // L):
                        sl = pl.ds(ck * L, L)
                        plsc.addupdate(acc.at[dl, sl], bufR[r, sl])
                    return 0

                lax.fori_loop(0, BATCH, row_body, 0)
                return 0

            lax.fori_loop(0, nb, batch_body, 0)

        pl.run_scoped(fast, pltpu.VMEM((BATCH, HID), jnp.float32))

        @pl.when(pend > pstart)
        def _():
            def slow(wmv, rowP, rowQ, rowS):
                pltpu.sync_copy(Wm, wmv)

                def paired_body(j, carry):
                    jb = (j // L) * L
                    jl = j - jb
                    pltpu.sync_copy(pp.at[pl.ds(jb, L)], sca.at[pl.ds(0, L)])
                    t_pos = _lane(sca, jl)
                    pltpu.sync_copy(plo.at[pl.ds(jb, L)], sca.at[pl.ds(0, L)])
                    r_lo = _lane(sca, jl)
                    pltpu.sync_copy(phi.at[pl.ds(jb, L)], sca.at[pl.ds(0, L)])
                    r_hi = _lane(sca, jl)
                    tb = (t_pos // L) * L
                    tl = t_pos - tb
                    pltpu.sync_copy(ss.at[pl.ds(tb, L)], sca.at[pl.ds(0, L)])
                    s_t = _lane(sca, tl)
                    pltpu.sync_copy(ee.at[pl.ds(tb, L)], sca.at[pl.ds(0, L)])
                    e_t = _lane(sca, tl)

                    for ck in range(HID // L):
                        rowS[pl.ds(ck * L, L)] = jnp.zeros((L,), jnp.float32)

                    def rev_body(r, carry2):
                        rb = (r // L) * L
                        rl = r - rb
                        pltpu.sync_copy(ss.at[pl.ds(rb, L)], sca.at[pl.ds(0, L)])
                        s_r = _lane(sca, rl)
                        pltpu.sync_copy(orr.at[pl.ds(rb, L)], sca.at[pl.ds(0, L)])
                        o_r = _lane(sca, rl)
                        pltpu.sync_copy(P.at[pl.ds(s_r, 1)], rowP)
                        pltpu.sync_copy(Q.at[pl.ds(o_r, 1)], rowQ)
                        for ck in range(HID // L):
                            sl = pl.ds(ck * L, L)
                            h = jnp.maximum(rowP[0, sl] + rowQ[0, sl], 0.0)
                            rowS[sl] = rowS[sl] + h
                        return carry2

                    lax.fori_loop(r_lo, r_hi, rev_body, 0)

                    def mv_body(k, accs):
                        sk = jnp.zeros((L,), jnp.float32) + _lane(rowS, k)
                        return tuple(
                            accs[ck] + sk * wmv[k, pl.ds(ck * L, L)]
                            for ck in range(HID // L)
                        )

                    accs0 = tuple(jnp.zeros((L,), jnp.float32)
                                  for _ in range(HID // L))
                    accs = lax.fori_loop(0, HID, mv_body, accs0)

                    pltpu.sync_copy(A_.at[pl.ds(s_t, 1)], rowP)
                    pltpu.sync_copy(Rm.at[pl.ds(s_t, 1)], rowQ)
                    dl = e_t - nbase
                    for ck in range(HID // L):
                        sl = pl.ds(ck * L, L)
                        delta = (jnp.maximum(rowP[0, sl] - accs[ck], 0.0)
                                 - rowQ[0, sl])
                        plsc.addupdate(acc.at[dl, sl], delta)
                    return carry

                lax.fori_loop(pstart, pend, paired_body, 0)

            pl.run_scoped(slow,
                          pltpu.VMEM((HID, HID), jnp.float32),
                          pltpu.VMEM((1, HID), jnp.float32),
                          pltpu.VMEM((1, HID), jnp.float32),
                          pltpu.VMEM((HID + L,), jnp.float32))

        pltpu.sync_copy(acc.at[pl.ds(0, SROWS)], out.at[pl.ds(nbase, SROWS)])


def _stage_d(Rm, A_, P, Q, Wm_T, ss_pad, order_pad, ee_pad, pp, plo, phi, meta,
             metb):
    mesh = plsc.VectorSubcoreMesh(core_axis_name="c", subcore_axis_name="s")
    return pl.kernel(
        _stage_d_body,
        out_type=jax.ShapeDtypeStruct((NPAD, HID), jnp.float32),
        mesh=mesh,
        scratch_types=[
            pltpu.VMEM((MLEN_A,), jnp.int32),
            pltpu.VMEM((MLEN_B,), jnp.int32),
            pltpu.VMEM((BATCH,), jnp.int32),
            pltpu.VMEM((BATCH,), jnp.int32),
            pltpu.VMEM((IDXLEN,), jnp.int32),
            pltpu.VMEM((SCALEN,), jnp.int32),
            pltpu.VMEM((ACC_D, HID), jnp.float32),
            pltpu.SemaphoreType.DMA,
            pltpu.SemaphoreType.DMA,
        ],
    )(Rm, A_, P, Q, Wm_T, ss_pad, order_pad, ee_pad, pp, plo, phi, meta, metb)


def kernel(node_feature, edge_featrue, edge_index, W_i, W_m, W_a, mpnn_hop):
    N = node_feature.shape[0]
    E = edge_index.shape[1]
    s = edge_index[0].astype(jnp.int32)
    e = edge_index[1].astype(jnp.int32)

    W_i1 = W_i[:, :ATOM].T
    W_i2 = W_i[:, ATOM:].T
    W_mT = W_m.T
    W_a1 = W_a[:, :ATOM].T
    W_a2 = W_a[:, ATOM:].T

    P = _matmul(node_feature, W_i1)
    Q = _matmul(edge_featrue, W_i2, block_rows=2000)

    t_iota = jnp.arange(E, dtype=jnp.int32)
    pk = e * 16384 + s
    pks, order = lax.sort((pk, t_iota), num_keys=1)
    ee = pks >> 14
    ss = pks & 16383
    rks = ee * N + ss
    fk = ss * N + ee

    tag_keys = jnp.concatenate([rks * 4 + 1, fk * 4 + 0, fk * 4 + 2])
    tag_ids = jnp.concatenate(
        [jnp.full((E,), 2 * E, jnp.int32), t_iota, t_iota + E])
    cks, cids = lax.sort((tag_keys, tag_ids), num_keys=1)
    is_data = ((cks & 3) == 1).astype(jnp.int32)
    c0 = jnp.cumsum(is_data) - is_data
    _, lohi = lax.sort((cids, c0), num_keys=1)
    lo = lohi[:E]
    hi = lohi[E:2 * E]
    paired = hi > lo

    pkey = jnp.where(paired, t_iota, t_iota + E)
    pkey_s, plo, phi, pe_s, pp = lax.sort((pkey, lo, hi, ee, t_iota),
                                          num_keys=1)
    pe = jnp.where(pkey_s < E, pe_s, NPAD)

    b32 = jnp.arange(33, dtype=jnp.int32) * TROWS
    b64 = jnp.arange(65, dtype=jnp.int32) * SROWS
    eb32 = jnp.sum(ee[None, :] < b32[:, None], axis=1).astype(jnp.int32)
    eb64 = jnp.sum(ee[None, :] < b64[:, None], axis=1).astype(jnp.int32)
    peb64 = jnp.sum(pe[None, :] < b64[:, None], axis=1).astype(jnp.int32)

    pad_i = jnp.zeros((128,), jnp.int32)
    ss_pad = jnp.concatenate([ss, pad_i])
    ee_pad = jnp.concatenate([ee, pad_i])
    order_pad = jnp.concatenate([order.astype(jnp.int32), pad_i])
    pp_pad = jnp.concatenate([pp, pad_i])
    plo_pad = jnp.concatenate([plo, pad_i])
    phi_pad = jnp.concatenate([phi, pad_i])

    meta = jnp.concatenate(
        [eb32, eb64, jnp.zeros((MLEN_A - 98,), jnp.int32)])
    metb = jnp.concatenate([peb64, jnp.zeros((MLEN_B - 65,), jnp.int32)])

    agg = _stage_b(P, Q, ss_pad, order_pad, ee_pad, meta)[:N]
    A_, Rm = _matmul_a_r(agg, W_mT)
    node_m = _stage_d(Rm, A_, P, Q, W_mT, ss_pad, order_pad, ee_pad,
                      pp_pad, plo_pad, phi_pad, meta, metb)[:N]

    return _final_stage(node_feature, node_m, W_a1, W_a2)
